# baseline (speedup 1.0000x reference)
import functools
import math

import jax
import jax.numpy as jnp
from jax import lax
from jax.experimental import pallas as pl
from jax.experimental.pallas import tpu as pltpu
from jax.experimental.pallas import tpu_sc as plsc

E = 8
K = 2
D = 768
H = 3072
S = 2048

T = 256
NB = (K * S) // T + E
NPAD = NB * T

NC = 2
NS = 16
NW = NC * NS
SLOTS = NPAD // NW
TOKS = S // NW

_INV_SQRT2 = 1.0 / math.sqrt(2.0)


def _gelu_exact(x):
    return 0.5 * x * (1.0 + jax.lax.erf(x * _INV_SQRT2))


def _router_body(x_ref, wr_ref, probs_ref, sel_ref, rw_ref, dest_ref, be_ref):
    x = x_ref[...]
    wr = wr_ref[...]
    logits = jax.lax.dot_general(wr, x, (((1,), (1,)), ((), ())),
                                 preferred_element_type=jnp.float32)
    m = jnp.max(logits, axis=0, keepdims=True)
    ex = jnp.exp(logits - m)
    probs = ex / jnp.sum(ex, axis=0, keepdims=True)
    probs_ref[...] = probs

    iota_e = jax.lax.broadcasted_iota(jnp.int32, (E, S), 0)
    m0 = jnp.max(probs, axis=0, keepdims=True)
    a0 = jnp.min(jnp.where(probs == m0, iota_e, E), axis=0, keepdims=True)
    masked = jnp.where(iota_e == a0, -jnp.inf, probs)
    m1 = jnp.max(masked, axis=0, keepdims=True)
    a1 = jnp.min(jnp.where(masked == m1, iota_e, E), axis=0, keepdims=True)

    denom = m0 + m1
    w0 = m0 / denom
    w1 = m1 / denom
    sel_ref[...] = jnp.concatenate([a0, a1], axis=0)
    rw_ref[...] = jnp.concatenate([w0, w1], axis=0)

    onehot = (jnp.where(iota_e == a0, 1.0, 0.0)
              + jnp.where(iota_e == a1, 1.0, 0.0))
    jr = jax.lax.broadcasted_iota(jnp.int32, (S, S), 0)
    ic = jax.lax.broadcasted_iota(jnp.int32, (S, S), 1)
    lt = jnp.where(jr < ic, 1.0, 0.0)
    ranks = jax.lax.dot_general(onehot, lt, (((1,), (0,)), ((), ())),
                                preferred_element_type=jnp.float32)
    counts = jnp.sum(onehot, axis=1, keepdims=True)
    nblk = jnp.floor((counts + (T - 1)) * (1.0 / T))
    er = jax.lax.broadcasted_iota(jnp.int32, (E, E), 0)
    ec = jax.lax.broadcasted_iota(jnp.int32, (E, E), 1)
    lt8 = jnp.where(ec < er, 1.0, 0.0)
    off_blk = jax.lax.dot_general(lt8, nblk, (((1,), (0,)), ((), ())),
                                  preferred_element_type=jnp.float32)
    po = off_blk * float(T)

    def slot_dest(a_k):
        sel_mask = iota_e == a_k
        rank_k = jnp.sum(jnp.where(sel_mask, ranks, 0.0), axis=0,
                         keepdims=True)
        po_k = jnp.sum(jnp.where(sel_mask, po, 0.0), axis=0,
                       keepdims=True)
        return (po_k + rank_k).astype(jnp.int32)

    dest_ref[...] = jnp.concatenate([slot_dest(a0), slot_dest(a1)], axis=0)

    b_iota = jax.lax.broadcasted_iota(jnp.int32, (E, NB), 1)
    ge = jnp.where(b_iota >= off_blk.astype(jnp.int32), 1, 0)
    be_ref[...] = jnp.sum(ge, axis=0, keepdims=True) - 1


HS = SLOTS // 2


def _dispatch_body(dest_hbm, rw_hbm, x_hbm, xg_hbm, w_hbm,
                   dest_v, rw_v, idx0, idx1, w0, w1, rows, sem):
    cid = lax.axis_index("c")
    sid = lax.axis_index("s")
    wid = sid * NC + cid
    lo = wid * SLOTS

    pltpu.sync_copy(dest_hbm, dest_v)
    pltpu.sync_copy(rw_hbm, rw_v)

    zi = jnp.zeros((16,), jnp.int32)
    zf = jnp.zeros((16,), jnp.float32)
    for j in range(HS // 16):
        idx0[pl.ds(j * 16, 16)] = zi
        idx1[pl.ds(j * 16, 16)] = zi
        w0[pl.ds(j * 16, 16)] = zf
        w1[pl.ds(j * 16, 16)] = zf

    def chunk(c, carry):
        toks = lax.iota(jnp.int32, 16) + c * 16
        for k in range(K):
            d = dest_v[k, pl.ds(c * 16, 16)]
            w = rw_v[k, pl.ds(c * 16, 16)]
            m0 = (d >= lo) & (d < lo + HS)
            m1 = (d >= lo + HS) & (d < lo + SLOTS)
            plsc.store_scatter(idx0, [d - lo], toks, mask=m0)
            plsc.store_scatter(w0, [d - lo], w, mask=m0)
            plsc.store_scatter(idx1, [d - (lo + HS)], toks, mask=m1)
            plsc.store_scatter(w1, [d - (lo + HS)], w, mask=m1)
        return carry

    lax.fori_loop(0, S // 16, chunk, 0)

    pltpu.sync_copy(w0, w_hbm.at[pl.ds(lo, HS)])
    pltpu.sync_copy(w1, w_hbm.at[pl.ds(lo + HS, HS)])
    for half, idx_buf in ((0, idx0), (1, idx1)):
        pltpu.async_copy(x_hbm.at[idx_buf], rows, sem).wait()
        pltpu.sync_copy(rows, xg_hbm.at[pl.ds(lo + half * HS, HS)])


def _ffn_body(be_ref, x_ref, w1_ref, b1_ref, w2_ref, b2_ref, w_ref, o_ref):
    xb = x_ref[...]
    hpre = jax.lax.dot_general(xb, w1_ref[0], (((1,), (1,)), ((), ())),
                               preferred_element_type=jnp.float32)
    hact = _gelu_exact(hpre + b1_ref[0])
    contrib = jax.lax.dot_general(hact, w2_ref[0], (((1,), (1,)), ((), ())),
                                  preferred_element_type=jnp.float32)
    wcol = w_ref[...]
    o_ref[...] = wcol * (contrib + b2_ref[0])


def _combine_body(dest_hbm, yg_hbm, y_hbm, d0_v, d1_v, buf0, buf1, sem0, sem1):
    cid = lax.axis_index("c")
    sid = lax.axis_index("s")
    wid = sid * NC + cid
    lo = wid * TOKS

    pltpu.sync_copy(dest_hbm.at[0, pl.ds(lo, TOKS)], d0_v)
    pltpu.sync_copy(dest_hbm.at[1, pl.ds(lo, TOKS)], d1_v)
    cp0 = pltpu.async_copy(yg_hbm.at[d0_v], buf0, sem0)
    cp1 = pltpu.async_copy(yg_hbm.at[d1_v], buf1, sem1)
    cp0.wait()
    cp1.wait()

    def row(i, carry):
        for j in range(D // 16):
            s = pl.ds(j * 16, 16)
            buf0[i, s] = buf0[i, s] + buf1[i, s]
        return carry

    lax.fori_loop(0, TOKS, row, 0)
    pltpu.sync_copy(buf0, y_hbm.at[pl.ds(lo, TOKS)])


def kernel(x, Wr, W1, b1, W2, b2):
    B = x.shape[0]
    x2 = x.reshape(B * S, D)

    probsT, selT, rwT, destT, be2 = pl.pallas_call(
        _router_body,
        out_shape=[
            jax.ShapeDtypeStruct((E, S), jnp.float32),
            jax.ShapeDtypeStruct((K, S), jnp.int32),
            jax.ShapeDtypeStruct((K, S), jnp.float32),
            jax.ShapeDtypeStruct((K, S), jnp.int32),
            jax.ShapeDtypeStruct((1, NB), jnp.int32),
        ],
    )(x2, Wr)

    mesh = plsc.VectorSubcoreMesh(core_axis_name="c", subcore_axis_name="s",
                                  num_cores=NC, num_subcores=NS)
    xg, slot_w = pl.kernel(
        _dispatch_body,
        out_type=[
            jax.ShapeDtypeStruct((NPAD, D), jnp.float32),
            jax.ShapeDtypeStruct((NPAD,), jnp.float32),
        ],
        mesh=mesh,
        scratch_types=[
            pltpu.VMEM((K, S), jnp.int32),
            pltpu.VMEM((K, S), jnp.float32),
            pltpu.VMEM((HS,), jnp.int32),
            pltpu.VMEM((HS,), jnp.int32),
            pltpu.VMEM((HS,), jnp.float32),
            pltpu.VMEM((HS,), jnp.float32),
            pltpu.VMEM((HS, D), jnp.float32),
            pltpu.SemaphoreType.DMA,
        ],
        compiler_params=pltpu.CompilerParams(needs_layout_passes=False,
                                             use_tc_tiling_on_sc=False),
    )(destT, rwT, x2)

    grid_spec = pltpu.PrefetchScalarGridSpec(
        num_scalar_prefetch=1,
        grid=(NB,),
        in_specs=[
            pl.BlockSpec((T, D), lambda b, be: (b, 0)),
            pl.BlockSpec((1, H, D), lambda b, be: (be[b], 0, 0)),
            pl.BlockSpec((1, 1, H), lambda b, be: (be[b], 0, 0)),
            pl.BlockSpec((1, D, H), lambda b, be: (be[b], 0, 0)),
            pl.BlockSpec((1, 1, D), lambda b, be: (be[b], 0, 0)),
            pl.BlockSpec((T, 1), lambda b, be: (b, 0)),
        ],
        out_specs=pl.BlockSpec((T, D), lambda b, be: (b, 0)),
    )
    yg = pl.pallas_call(
        _ffn_body,
        grid_spec=grid_spec,
        out_shape=jax.ShapeDtypeStruct((NPAD, D), jnp.float32),
    )(be2.reshape(NB), xg, W1, b1.reshape(E, 1, H), W2, b2.reshape(E, 1, D),
      slot_w.reshape(NPAD, 1))

    y = pl.kernel(
        _combine_body,
        out_type=jax.ShapeDtypeStruct((B * S, D), jnp.float32),
        mesh=mesh,
        scratch_types=[
            pltpu.VMEM((TOKS,), jnp.int32),
            pltpu.VMEM((TOKS,), jnp.int32),
            pltpu.VMEM((TOKS, D), jnp.float32),
            pltpu.VMEM((TOKS, D), jnp.float32),
            pltpu.SemaphoreType.DMA,
            pltpu.SemaphoreType.DMA,
        ],
        compiler_params=pltpu.CompilerParams(needs_layout_passes=False),
    )(destT, yg)

    expert_outputs = y.reshape(B, S, D)
    routing_probs = probsT.T.reshape(B, S, E)
    selected_experts = selT.T.reshape(B, S, K)
    routing_weights = rwT.T.reshape(B, S, K)
    return (expert_outputs, routing_probs, selected_experts, routing_weights)


# speedup vs baseline: 1.1075x; 1.1075x over previous
import functools
import math

import jax
import jax.numpy as jnp
from jax import lax
from jax.experimental import pallas as pl
from jax.experimental.pallas import tpu as pltpu
from jax.experimental.pallas import tpu_sc as plsc

E = 8
K = 2
D = 768
H = 3072
S = 2048

T = 256
NB = (K * S) // T + E
NPAD = NB * T

NC = 2
NS = 16
NW = NC * NS
SLOTS = NPAD // NW
TOKS = S // NW

_INV_SQRT2 = 1.0 / math.sqrt(2.0)


def _gelu_exact(x):
    return 0.5 * x * (1.0 + jax.lax.erf(x * _INV_SQRT2))


def _router_body(x_ref, wr_ref, probs_ref, sel_ref, rw_ref, dest_ref, be_ref):
    x = x_ref[...]
    wr = wr_ref[...]
    logits = jax.lax.dot_general(wr, x, (((1,), (1,)), ((), ())),
                                 preferred_element_type=jnp.float32)
    m = jnp.max(logits, axis=0, keepdims=True)
    ex = jnp.exp(logits - m)
    probs = ex / jnp.sum(ex, axis=0, keepdims=True)
    probs_ref[...] = probs

    iota_e = jax.lax.broadcasted_iota(jnp.int32, (E, S), 0)
    m0 = jnp.max(probs, axis=0, keepdims=True)
    a0 = jnp.min(jnp.where(probs == m0, iota_e, E), axis=0, keepdims=True)
    masked = jnp.where(iota_e == a0, -jnp.inf, probs)
    m1 = jnp.max(masked, axis=0, keepdims=True)
    a1 = jnp.min(jnp.where(masked == m1, iota_e, E), axis=0, keepdims=True)

    denom = m0 + m1
    w0 = m0 / denom
    w1 = m1 / denom
    sel_ref[...] = jnp.concatenate([a0, a1], axis=0)
    rw_ref[...] = jnp.concatenate([w0, w1], axis=0)

    onehot = (jnp.where(iota_e == a0, 1.0, 0.0)
              + jnp.where(iota_e == a1, 1.0, 0.0))
    jr = jax.lax.broadcasted_iota(jnp.int32, (S, S), 0)
    ic = jax.lax.broadcasted_iota(jnp.int32, (S, S), 1)
    lt = jnp.where(jr < ic, 1.0, 0.0)
    ranks = jax.lax.dot_general(onehot, lt, (((1,), (0,)), ((), ())),
                                preferred_element_type=jnp.float32)
    counts = jnp.sum(onehot, axis=1, keepdims=True)
    nblk = jnp.floor((counts + (T - 1)) * (1.0 / T))
    er = jax.lax.broadcasted_iota(jnp.int32, (E, E), 0)
    ec = jax.lax.broadcasted_iota(jnp.int32, (E, E), 1)
    lt8 = jnp.where(ec < er, 1.0, 0.0)
    off_blk = jax.lax.dot_general(lt8, nblk, (((1,), (0,)), ((), ())),
                                  preferred_element_type=jnp.float32)
    po = off_blk * float(T)

    def slot_dest(a_k):
        sel_mask = iota_e == a_k
        rank_k = jnp.sum(jnp.where(sel_mask, ranks, 0.0), axis=0,
                         keepdims=True)
        po_k = jnp.sum(jnp.where(sel_mask, po, 0.0), axis=0,
                       keepdims=True)
        return (po_k + rank_k).astype(jnp.int32)

    dest_ref[...] = jnp.concatenate([slot_dest(a0), slot_dest(a1)], axis=0)

    b_iota = jax.lax.broadcasted_iota(jnp.int32, (E, NB), 1)
    ge = jnp.where(b_iota >= off_blk.astype(jnp.int32), 1, 0)
    be_ref[...] = jnp.sum(ge, axis=0, keepdims=True) - 1


HS = SLOTS // 2


def _dispatch_body(dest_hbm, rw_hbm, x_hbm, xg_hbm, w_hbm,
                   dest_v, rw_v, idx0, idx1, w0, w1, rows, rows1, sem, sem1):
    cid = lax.axis_index("c")
    sid = lax.axis_index("s")
    wid = sid * NC + cid
    lo = wid * SLOTS

    pltpu.sync_copy(dest_hbm, dest_v)
    pltpu.sync_copy(rw_hbm, rw_v)

    zi = jnp.zeros((16,), jnp.int32)
    zf = jnp.zeros((16,), jnp.float32)
    for j in range(HS // 16):
        idx0[pl.ds(j * 16, 16)] = zi
        idx1[pl.ds(j * 16, 16)] = zi
        w0[pl.ds(j * 16, 16)] = zf
        w1[pl.ds(j * 16, 16)] = zf

    def chunk(c, carry):
        toks = lax.iota(jnp.int32, 16) + c * 16
        for k in range(K):
            d = dest_v[k, pl.ds(c * 16, 16)]
            w = rw_v[k, pl.ds(c * 16, 16)]
            m0 = (d >= lo) & (d < lo + HS)
            m1 = (d >= lo + HS) & (d < lo + SLOTS)
            plsc.store_scatter(idx0, [d - lo], toks, mask=m0)
            plsc.store_scatter(w0, [d - lo], w, mask=m0)
            plsc.store_scatter(idx1, [d - (lo + HS)], toks, mask=m1)
            plsc.store_scatter(w1, [d - (lo + HS)], w, mask=m1)
        return carry

    lax.fori_loop(0, S // 16, chunk, 0)

    pltpu.sync_copy(w0, w_hbm.at[pl.ds(lo, HS)])
    pltpu.sync_copy(w1, w_hbm.at[pl.ds(lo + HS, HS)])
    QS = HS // 2
    bufs = (rows, rows1)
    sems = (sem, sem1)
    idxs = ((idx0, 0), (idx0, QS), (idx1, 0), (idx1, QS))
    cps = [None, None]
    for q, (ib, off) in enumerate(idxs):
        cps[q % 2] = pltpu.async_copy(x_hbm.at[ib.at[pl.ds(off, QS)]],
                                      bufs[q % 2], sems[q % 2])
        if q >= 1:
            prev = q - 1
            cps[prev % 2].wait()
            pltpu.sync_copy(bufs[prev % 2],
                            xg_hbm.at[pl.ds(lo + prev * QS, QS)])
    cps[3 % 2].wait()
    pltpu.sync_copy(bufs[3 % 2], xg_hbm.at[pl.ds(lo + 3 * QS, QS)])


def _ffn_body(be_ref, x_ref, w1_ref, b1_ref, w2_ref, b2_ref, w_ref, o_ref):
    xb = x_ref[...]
    hpre = jax.lax.dot_general(xb, w1_ref[0], (((1,), (1,)), ((), ())),
                               preferred_element_type=jnp.float32)
    hact = _gelu_exact(hpre + b1_ref[0])
    contrib = jax.lax.dot_general(hact, w2_ref[0], (((1,), (1,)), ((), ())),
                                  preferred_element_type=jnp.float32)
    wcol = w_ref[...]
    o_ref[...] = wcol * (contrib + b2_ref[0])


def _combine_body(dest_hbm, yg_hbm, y_hbm, d0_v, d1_v, buf0, buf1, sem0, sem1):
    cid = lax.axis_index("c")
    sid = lax.axis_index("s")
    wid = sid * NC + cid
    lo = wid * TOKS

    pltpu.sync_copy(dest_hbm.at[0, pl.ds(lo, TOKS)], d0_v)
    pltpu.sync_copy(dest_hbm.at[1, pl.ds(lo, TOKS)], d1_v)
    cp0 = pltpu.async_copy(yg_hbm.at[d0_v], buf0, sem0)
    cp1 = pltpu.async_copy(yg_hbm.at[d1_v], buf1, sem1)
    cp0.wait()
    cp1.wait()

    def row(i, carry):
        for j in range(D // 16):
            s = pl.ds(j * 16, 16)
            buf0[i, s] = buf0[i, s] + buf1[i, s]
        return carry

    lax.fori_loop(0, TOKS, row, 0)
    pltpu.sync_copy(buf0, y_hbm.at[pl.ds(lo, TOKS)])


def kernel(x, Wr, W1, b1, W2, b2):
    B = x.shape[0]
    x2 = x.reshape(B * S, D)

    probsT, selT, rwT, destT, be2 = pl.pallas_call(
        _router_body,
        out_shape=[
            jax.ShapeDtypeStruct((E, S), jnp.float32),
            jax.ShapeDtypeStruct((K, S), jnp.int32),
            jax.ShapeDtypeStruct((K, S), jnp.float32),
            jax.ShapeDtypeStruct((K, S), jnp.int32),
            jax.ShapeDtypeStruct((1, NB), jnp.int32),
        ],
    )(x2, Wr)

    mesh = plsc.VectorSubcoreMesh(core_axis_name="c", subcore_axis_name="s",
                                  num_cores=NC, num_subcores=NS)
    xg, slot_w = pl.kernel(
        _dispatch_body,
        out_type=[
            jax.ShapeDtypeStruct((NPAD, D), jnp.float32),
            jax.ShapeDtypeStruct((NPAD,), jnp.float32),
        ],
        mesh=mesh,
        scratch_types=[
            pltpu.VMEM((K, S), jnp.int32),
            pltpu.VMEM((K, S), jnp.float32),
            pltpu.VMEM((HS,), jnp.int32),
            pltpu.VMEM((HS,), jnp.int32),
            pltpu.VMEM((HS,), jnp.float32),
            pltpu.VMEM((HS,), jnp.float32),
            pltpu.VMEM((HS // 2, D), jnp.float32),
            pltpu.VMEM((HS // 2, D), jnp.float32),
            pltpu.SemaphoreType.DMA,
            pltpu.SemaphoreType.DMA,
        ],
        compiler_params=pltpu.CompilerParams(needs_layout_passes=False),
    )(destT, rwT, x2)

    grid_spec = pltpu.PrefetchScalarGridSpec(
        num_scalar_prefetch=1,
        grid=(NB,),
        in_specs=[
            pl.BlockSpec((T, D), lambda b, be: (b, 0)),
            pl.BlockSpec((1, H, D), lambda b, be: (be[b], 0, 0)),
            pl.BlockSpec((1, 1, H), lambda b, be: (be[b], 0, 0)),
            pl.BlockSpec((1, D, H), lambda b, be: (be[b], 0, 0)),
            pl.BlockSpec((1, 1, D), lambda b, be: (be[b], 0, 0)),
            pl.BlockSpec((T, 1), lambda b, be: (b, 0)),
        ],
        out_specs=pl.BlockSpec((T, D), lambda b, be: (b, 0)),
    )
    yg = pl.pallas_call(
        _ffn_body,
        grid_spec=grid_spec,
        out_shape=jax.ShapeDtypeStruct((NPAD, D), jnp.float32),
    )(be2.reshape(NB), xg, W1, b1.reshape(E, 1, H), W2, b2.reshape(E, 1, D),
      slot_w.reshape(NPAD, 1))

    y = pl.kernel(
        _combine_body,
        out_type=jax.ShapeDtypeStruct((B * S, D), jnp.float32),
        mesh=mesh,
        scratch_types=[
            pltpu.VMEM((TOKS,), jnp.int32),
            pltpu.VMEM((TOKS,), jnp.int32),
            pltpu.VMEM((TOKS, D), jnp.float32),
            pltpu.VMEM((TOKS, D), jnp.float32),
            pltpu.SemaphoreType.DMA,
            pltpu.SemaphoreType.DMA,
        ],
        compiler_params=pltpu.CompilerParams(needs_layout_passes=False),
    )(destT, yg)

    expert_outputs = y.reshape(B, S, D)
    routing_probs = probsT.T.reshape(B, S, E)
    selected_experts = selT.T.reshape(B, S, K)
    routing_weights = rwT.T.reshape(B, S, K)
    return (expert_outputs, routing_probs, selected_experts, routing_weights)


# speedup vs baseline: 1.5493x; 1.3989x over previous
import functools
import math

import jax
import jax.numpy as jnp
from jax import lax
from jax.experimental import pallas as pl
from jax.experimental.pallas import tpu as pltpu
from jax.experimental.pallas import tpu_sc as plsc

E = 8
K = 2
D = 768
H = 3072
S = 2048

T = 256
NB = (K * S) // T + E
NPAD = NB * T

NC = 2
NS = 16
NW = NC * NS
SLOTS = NPAD // NW
TOKS = S // NW

_INV_SQRT2 = 1.0 / math.sqrt(2.0)


def _gelu_exact(x):
    return 0.5 * x * (1.0 + jax.lax.erf(x * _INV_SQRT2))


def _router_body(x_ref, wr_ref, probs_ref, sel_ref, rw_ref, dest_ref, be_ref):
    x = x_ref[...]
    wr = wr_ref[...]
    logits = jax.lax.dot_general(wr, x, (((1,), (1,)), ((), ())),
                                 preferred_element_type=jnp.float32)
    m = jnp.max(logits, axis=0, keepdims=True)
    ex = jnp.exp(logits - m)
    probs = ex / jnp.sum(ex, axis=0, keepdims=True)
    probs_ref[...] = probs

    iota_e = jax.lax.broadcasted_iota(jnp.int32, (E, S), 0)
    m0 = jnp.max(probs, axis=0, keepdims=True)
    a0 = jnp.min(jnp.where(probs == m0, iota_e, E), axis=0, keepdims=True)
    masked = jnp.where(iota_e == a0, -jnp.inf, probs)
    m1 = jnp.max(masked, axis=0, keepdims=True)
    a1 = jnp.min(jnp.where(masked == m1, iota_e, E), axis=0, keepdims=True)

    denom = m0 + m1
    w0 = m0 / denom
    w1 = m1 / denom
    sel_ref[...] = jnp.concatenate([a0, a1], axis=0)
    rw_ref[...] = jnp.concatenate([w0, w1], axis=0)

    onehot = (jnp.where(iota_e == a0, 1.0, 0.0)
              + jnp.where(iota_e == a1, 1.0, 0.0))
    jr = jax.lax.broadcasted_iota(jnp.int32, (S, S), 0)
    ic = jax.lax.broadcasted_iota(jnp.int32, (S, S), 1)
    lt = jnp.where(jr < ic, 1.0, 0.0)
    ranks = jax.lax.dot_general(onehot, lt, (((1,), (0,)), ((), ())),
                                preferred_element_type=jnp.float32)
    counts = jnp.sum(onehot, axis=1, keepdims=True)
    nblk = jnp.floor((counts + (T - 1)) * (1.0 / T))
    er = jax.lax.broadcasted_iota(jnp.int32, (E, E), 0)
    ec = jax.lax.broadcasted_iota(jnp.int32, (E, E), 1)
    lt8 = jnp.where(ec < er, 1.0, 0.0)
    off_blk = jax.lax.dot_general(lt8, nblk, (((1,), (0,)), ((), ())),
                                  preferred_element_type=jnp.float32)
    po = off_blk * float(T)

    def slot_dest(a_k):
        sel_mask = iota_e == a_k
        rank_k = jnp.sum(jnp.where(sel_mask, ranks, 0.0), axis=0,
                         keepdims=True)
        po_k = jnp.sum(jnp.where(sel_mask, po, 0.0), axis=0,
                       keepdims=True)
        return (po_k + rank_k).astype(jnp.int32)

    dest_ref[...] = jnp.concatenate([slot_dest(a0), slot_dest(a1)], axis=0)

    b_iota = jax.lax.broadcasted_iota(jnp.int32, (E, NB), 1)
    ge = jnp.where(b_iota >= off_blk.astype(jnp.int32), 1, 0)
    be_ref[...] = jnp.sum(ge, axis=0, keepdims=True) - 1


def _permute_body(dest_ref, x_ref, xg_ref):
    b = pl.program_id(0)
    d0 = dest_ref[0:1, :]
    d1 = dest_ref[1:2, :]
    p_iota = jax.lax.broadcasted_iota(jnp.int32, (T, S), 0) + b * T
    perm = (jnp.where(d0 == p_iota, 1.0, 0.0)
            + jnp.where(d1 == p_iota, 1.0, 0.0))
    xg_ref[...] = jax.lax.dot_general(
        perm, x_ref[...], (((1,), (0,)), ((), ())),
        preferred_element_type=jnp.float32)


def _ffn_body(be_ref, x_ref, w1_ref, b1_ref, w2_ref, b2_ref, o_ref):
    xb = x_ref[...]
    hpre = jax.lax.dot_general(xb, w1_ref[0], (((1,), (1,)), ((), ())),
                               preferred_element_type=jnp.float32)
    hact = _gelu_exact(hpre + b1_ref[0])
    contrib = jax.lax.dot_general(hact, w2_ref[0], (((1,), (1,)), ((), ())),
                                  preferred_element_type=jnp.float32)
    o_ref[...] = contrib + b2_ref[0]


def _combine_body(dest_hbm, rw_hbm, yg_hbm, y_hbm, d0_v, d1_v, w0_v, w1_v,
                  buf0, buf1, sem0, sem1):
    cid = lax.axis_index("c")
    sid = lax.axis_index("s")
    wid = sid * NC + cid
    lo = wid * TOKS

    pltpu.sync_copy(dest_hbm.at[0, pl.ds(lo, TOKS)], d0_v)
    pltpu.sync_copy(dest_hbm.at[1, pl.ds(lo, TOKS)], d1_v)
    pltpu.sync_copy(rw_hbm.at[0, pl.ds(lo, TOKS)], w0_v)
    pltpu.sync_copy(rw_hbm.at[1, pl.ds(lo, TOKS)], w1_v)
    cp0 = pltpu.async_copy(yg_hbm.at[d0_v], buf0, sem0)
    cp1 = pltpu.async_copy(yg_hbm.at[d1_v], buf1, sem1)
    cp0.wait()
    cp1.wait()

    def group(g, carry):
        wv0 = w0_v[pl.ds(g * 16, 16)]
        wv1 = w1_v[pl.ds(g * 16, 16)]
        for r in range(16):
            i = g * 16 + r
            w0s = wv0[r]
            w1s = wv1[r]
            for j in range(D // 16):
                s = pl.ds(j * 16, 16)
                buf0[i, s] = buf0[i, s] * w0s + buf1[i, s] * w1s
        return carry

    lax.fori_loop(0, TOKS // 16, group, 0)
    pltpu.sync_copy(buf0, y_hbm.at[pl.ds(lo, TOKS)])


def kernel(x, Wr, W1, b1, W2, b2):
    B = x.shape[0]
    x2 = x.reshape(B * S, D)

    probsT, selT, rwT, destT, be2 = pl.pallas_call(
        _router_body,
        out_shape=[
            jax.ShapeDtypeStruct((E, S), jnp.float32),
            jax.ShapeDtypeStruct((K, S), jnp.int32),
            jax.ShapeDtypeStruct((K, S), jnp.float32),
            jax.ShapeDtypeStruct((K, S), jnp.int32),
            jax.ShapeDtypeStruct((1, NB), jnp.int32),
        ],
    )(x2, Wr)

    mesh = plsc.VectorSubcoreMesh(core_axis_name="c", subcore_axis_name="s",
                                  num_cores=NC, num_subcores=NS)
    xg = pl.pallas_call(
        _permute_body,
        grid=(NB,),
        in_specs=[
            pl.BlockSpec((K, S), lambda b: (0, 0)),
            pl.BlockSpec((S, D), lambda b: (0, 0)),
        ],
        out_specs=pl.BlockSpec((T, D), lambda b: (b, 0)),
        out_shape=jax.ShapeDtypeStruct((NPAD, D), jnp.float32),
    )(destT, x2)

    grid_spec = pltpu.PrefetchScalarGridSpec(
        num_scalar_prefetch=1,
        grid=(NB,),
        in_specs=[
            pl.BlockSpec((T, D), lambda b, be: (b, 0)),
            pl.BlockSpec((1, H, D), lambda b, be: (be[b], 0, 0)),
            pl.BlockSpec((1, 1, H), lambda b, be: (be[b], 0, 0)),
            pl.BlockSpec((1, D, H), lambda b, be: (be[b], 0, 0)),
            pl.BlockSpec((1, 1, D), lambda b, be: (be[b], 0, 0)),
        ],
        out_specs=pl.BlockSpec((T, D), lambda b, be: (b, 0)),
    )
    yg = pl.pallas_call(
        _ffn_body,
        grid_spec=grid_spec,
        out_shape=jax.ShapeDtypeStruct((NPAD, D), jnp.float32),
    )(be2.reshape(NB), xg, W1, b1.reshape(E, 1, H), W2, b2.reshape(E, 1, D))

    y = pl.kernel(
        _combine_body,
        out_type=jax.ShapeDtypeStruct((B * S, D), jnp.float32),
        mesh=mesh,
        scratch_types=[
            pltpu.VMEM((TOKS,), jnp.int32),
            pltpu.VMEM((TOKS,), jnp.int32),
            pltpu.VMEM((TOKS,), jnp.float32),
            pltpu.VMEM((TOKS,), jnp.float32),
            pltpu.VMEM((TOKS, D), jnp.float32),
            pltpu.VMEM((TOKS, D), jnp.float32),
            pltpu.SemaphoreType.DMA,
            pltpu.SemaphoreType.DMA,
        ],
        compiler_params=pltpu.CompilerParams(needs_layout_passes=False),
    )(destT, rwT, yg)

    expert_outputs = y.reshape(B, S, D)
    routing_probs = probsT.T.reshape(B, S, E)
    selected_experts = selT.T.reshape(B, S, K)
    routing_weights = rwT.T.reshape(B, S, K)
    return (expert_outputs, routing_probs, selected_experts, routing_weights)
